# baseline (speedup 1.0000x reference)
import jax
import jax.numpy as jnp
from jax import lax
from jax.experimental import pallas as pl
from jax.experimental.pallas import tpu as pltpu
from jax.experimental.pallas import tpu_sc as plsc

_NC, _NS = 2, 16
_NW = _NC * _NS

_B, _H, _D, _V = 16384, 50, 32, 1000000
_SPW = _B // _NW
_LT = _SPW // 128
_DR = _D // 8


def _make_gather():
    mesh = plsc.VectorSubcoreMesh(
        core_axis_name="c", subcore_axis_name="s",
        num_cores=_NC, num_subcores=_NS,
    )

    def body(el_hbm, table_hbm, out_hbm, idx_v, idx_h, rows, tiles,
             g0, g1, w0, w1):
        gsem = (g0, g1)
        wsem = (w0, w1)
        wid = lax.axis_index("s") * _NC + lax.axis_index("c")
        base = wid * _SPW
        pltpu.sync_copy(el_hbm.at[pl.ds(base * _H, _SPW * _H)], idx_v)
        lanes = lax.iota(jnp.int32, 16)

        def build_idx(h, b):
            for k in range(_SPW // 16):
                offs = (k * 16 + lanes) * _H + h
                idx_h[b, pl.ds(k * 16, 16)] = plsc.load_gather(idx_v, [offs])

        def gather_desc(b):
            return pltpu.make_async_copy(
                table_hbm.at[idx_h.at[b]], rows.at[b], gsem[b])

        def start_gather(b):
            pltpu.async_copy(table_hbm.at[idx_h.at[b]], rows.at[b], gsem[b])

        def out_slab(h):
            return out_hbm.at[pl.ds(h, 1), pl.ds(0, _DR), pl.ds(wid * _LT, _LT)]

        def transpose(b):
            def tdr(dr, carry):
                for q in range(_LT):
                    for s in range(8):
                        d = dr * 8 + s
                        dvec = lanes * 0 + d
                        for l0 in range(8):
                            j = q * 128 + l0 * 16 + lanes
                            v = plsc.load_gather(rows.at[b], [j, dvec])
                            tiles[b, 0, dr, q, s, pl.ds(l0 * 16, 16)] = v
                return carry
            lax.fori_loop(0, _DR, tdr, 0)

        build_idx(0, 0)
        start_gather(0)
        build_idx(1, 1)
        start_gather(1)

        def step(hh, carry):
            for b in range(2):
                h = 2 * hh + b
                gather_desc(b).wait()

                @pl.when(hh > 0)
                def _():
                    pltpu.make_async_copy(
                        out_slab(h - 2), tiles.at[b], wsem[b]).wait()

                transpose(b)
                pltpu.async_copy(tiles.at[b], out_slab(h), wsem[b])

                @pl.when(h + 2 < _H)
                def _():
                    build_idx(h + 2, b)
                    start_gather(b)
            return carry

        lax.fori_loop(0, _H // 2, step, 0)
        pltpu.make_async_copy(out_slab(_H - 2), tiles.at[0], wsem[0]).wait()
        pltpu.make_async_copy(out_slab(_H - 1), tiles.at[1], wsem[1]).wait()

    return pl.kernel(
        body,
        out_type=jax.ShapeDtypeStruct((_H, _DR, 128, 8, 128), jnp.float32),
        mesh=mesh,
        scratch_types=[
            pltpu.VMEM((_SPW * _H,), jnp.int32),
            pltpu.VMEM((2, _SPW), jnp.int32),
            pltpu.VMEM((2, _SPW, _D), jnp.float32),
            pltpu.VMEM((2, 1, _DR, _LT, 8, 128), jnp.float32),
            pltpu.SemaphoreType.DMA,
            pltpu.SemaphoreType.DMA,
            pltpu.SemaphoreType.DMA,
            pltpu.SemaphoreType.DMA,
        ],
        compiler_params=pltpu.CompilerParams(use_tc_tiling_on_sc=False, needs_layout_passes=False),
    )


def kernel(element, table):
    flat = element.reshape(_B * _H)
    out5 = _make_gather()(flat, table)
    return out5.transpose((2, 4, 0, 1, 3)).reshape(_B, _H, _D)


# speedup vs baseline: 1.8299x; 1.8299x over previous
import jax
import jax.numpy as jnp
from jax import lax
from jax.experimental import pallas as pl
from jax.experimental.pallas import tpu as pltpu
from jax.experimental.pallas import tpu_sc as plsc

_NC, _NS = 2, 16
_NW = _NC * _NS

_B, _H, _D, _V = 16384, 50, 32, 1000000
_SPW = _B // _NW
_LT = _SPW // 128
_DR = _D // 8


def _make_gather():
    mesh = plsc.VectorSubcoreMesh(
        core_axis_name="c", subcore_axis_name="s",
        num_cores=_NC, num_subcores=_NS,
    )

    def body(el_hbm, table_hbm, out_hbm, idx_v, idx_h, rows, tiles,
             g0, g1, w0, w1):
        gsem = (g0, g1)
        wsem = (w0, w1)
        wid = lax.axis_index("s") * _NC + lax.axis_index("c")
        base = wid * _SPW
        pltpu.sync_copy(el_hbm.at[pl.ds(base * _H, _SPW * _H)], idx_v)
        lanes = lax.iota(jnp.int32, 16)

        def build_idx(h, b):
            for k in range(_SPW // 16):
                offs = (k * 16 + lanes) * _H + h
                idx_h[b, pl.ds(k * 16, 16)] = plsc.load_gather(idx_v, [offs])

        def gather_desc(b):
            return pltpu.make_async_copy(
                table_hbm.at[idx_h.at[b]], rows.at[b], gsem[b])

        def start_gather(b):
            pltpu.async_copy(table_hbm.at[idx_h.at[b]], rows.at[b], gsem[b])

        def out_slab(h):
            return out_hbm.at[pl.ds(h, 1), pl.ds(0, _DR), pl.ds(wid * _LT, _LT)]

        def tiles_view(b):
            return tiles.at[b, pl.ds(0, 1), pl.ds(0, _DR), pl.ds(0, _LT),
                            pl.ds(0, 8), pl.ds(0, 128)]

        def write_desc(b, h):
            return pltpu.make_async_copy(out_slab(h), tiles_view(b), wsem[b])

        dr_lo = lanes >> 3
        dr_hi = dr_lo + 2
        s_vec = lanes & 7
        zero16 = lanes * 0

        def transpose(b):
            def tj(jb, carry):
                for jj in range(16):
                    j = jb * 16 + jj
                    q_vec = zero16 + (j >> 7)
                    l_vec = zero16 + (j & 127)
                    v0 = rows[b, j, pl.ds(0, 16)]
                    v1 = rows[b, j, pl.ds(16, 16)]
                    plsc.store_scatter(
                        tiles.at[b, 0], [dr_lo, q_vec, s_vec, l_vec], v0)
                    plsc.store_scatter(
                        tiles.at[b, 0], [dr_hi, q_vec, s_vec, l_vec], v1)
                return carry
            lax.fori_loop(0, _SPW // 16, tj, 0)

        build_idx(0, 0)
        start_gather(0)
        build_idx(1, 1)
        start_gather(1)

        def step(hh, carry):
            for b in range(2):
                h = 2 * hh + b
                gather_desc(b).wait()

                @pl.when(hh > 0)
                def _():
                    write_desc(b, h - 2).wait()

                transpose(b)
                pltpu.async_copy(tiles_view(b), out_slab(h), wsem[b])

                @pl.when(h + 2 < _H)
                def _():
                    build_idx(h + 2, b)
                    start_gather(b)
            return carry

        lax.fori_loop(0, _H // 2, step, 0)
        write_desc(0, _H - 2).wait()
        write_desc(1, _H - 1).wait()

    return pl.kernel(
        body,
        out_type=jax.ShapeDtypeStruct((_H, _DR, 128, 8, 128), jnp.float32),
        mesh=mesh,
        scratch_types=[
            pltpu.VMEM((_SPW * _H,), jnp.int32),
            pltpu.VMEM((2, _SPW), jnp.int32),
            pltpu.VMEM((2, _SPW, _D), jnp.float32),
            pltpu.VMEM((2, 1, _DR, _LT, 10, 129), jnp.float32),
            pltpu.SemaphoreType.DMA,
            pltpu.SemaphoreType.DMA,
            pltpu.SemaphoreType.DMA,
            pltpu.SemaphoreType.DMA,
        ],
        compiler_params=pltpu.CompilerParams(use_tc_tiling_on_sc=False, needs_layout_passes=False),
    )


def kernel(element, table):
    flat = element.reshape(_B * _H)
    out5 = _make_gather()(flat, table)
    return out5.transpose((2, 4, 0, 1, 3)).reshape(_B, _H, _D)
